# Initial kernel scaffold; baseline (speedup 1.0000x reference)
import jax
import jax.numpy as jnp
from jax.experimental import pallas as pl


def kernel(bone_features, bone_adj, W1, b1, W2, b2, W3, b3):
    raise NotImplementedError("write your pallas kernel here")


# speedup vs baseline: 4797.5355x; 4797.5355x over previous
import jax
import jax.numpy as jnp
from jax.experimental import pallas as pl


def _gcn3_kernel(adj_ref, x_ref, w1_ref, b1_ref, w2_ref, b2_ref, w3_ref,
                 b3_ref, out_ref):
    S = (adj_ref[...] != 0).astype(jnp.float32)
    n = S.shape[0]
    ones = jnp.ones((n, 1), jnp.float32)
    deg = 1.0 + jax.lax.dot_general(
        S, ones, (((0,), (0,)), ((), ())), preferred_element_type=jnp.float32)
    dis = jax.lax.rsqrt(deg)
    dis2 = dis * dis

    x = x_ref[...]
    for w_ref, b_ref in ((w1_ref, b1_ref), (w2_ref, b2_ref), (w3_ref, b3_ref)):
        h = jnp.dot(x, w_ref[...], preferred_element_type=jnp.float32)
        y = dis * h
        agg = jax.lax.dot_general(
            S, y, (((0,), (0,)), ((), ())), preferred_element_type=jnp.float32)
        z = dis * agg + dis2 * h
        x = jnp.maximum(z + b_ref[...], 0.0)
    out_ref[...] = x


def kernel(bone_features, bone_adj, W1, b1, W2, b2, W3, b3):
    n, d = bone_features.shape
    return pl.pallas_call(
        _gcn3_kernel,
        out_shape=jax.ShapeDtypeStruct((n, W3.shape[1]), jnp.float32),
    )(bone_adj, bone_features,
      W1, b1.reshape(1, -1), W2, b2.reshape(1, -1), W3, b3.reshape(1, -1))
